# Initial kernel scaffold; baseline (speedup 1.0000x reference)
import jax
import jax.numpy as jnp
from jax.experimental import pallas as pl


def kernel(outputs, labels, global_logit):
    raise NotImplementedError("write your pallas kernel here")


# speedup vs baseline: 2.0837x; 2.0837x over previous
import functools

import jax
import jax.numpy as jnp
from jax import lax
from jax.experimental import pallas as pl
from jax.experimental.pallas import tpu as pltpu

N = 16384
D = 128
G = 100
BLK = 512
NB = N // BLK
EPS = 1e-8
NEG = -1e30
HI_BITS = 0x40800000


def _body(out_ref, lbl_ref, g_ref, loss_ref, keys_s, nll_s, accf, acci):
    i = pl.program_id(0)

    @pl.when(i == 0)
    def _init():
        accf[0] = 0.0
        acci[0] = 0
        acci[1] = 0

    x = out_ref[:, :]
    g = g_ref[:, :]

    gss = jnp.sum(g * g, axis=1, keepdims=True)
    gn = g / jnp.maximum(jnp.sqrt(gss), EPS)

    xss = jnp.sum(x * x, axis=1, keepdims=True)
    xn = x / jnp.maximum(jnp.sqrt(xss), EPS)

    sims = lax.dot_general(xn, gn, (((1,), (1,)), ((), ())),
                           preferred_element_type=jnp.float32)
    col = lax.broadcasted_iota(jnp.int32, (BLK, D), 1)
    simsm = jnp.where(col < G, sims, NEG)

    top1 = jnp.max(simsm, axis=1, keepdims=True)
    arg1 = jnp.min(jnp.where(simsm == top1, col, D), axis=1, keepdims=True)
    top2 = jnp.max(jnp.where(col == arg1, NEG, simsm), axis=1, keepdims=True)
    diff = top1 - top2

    m = jnp.max(x, axis=1, keepdims=True)
    lse = m + jnp.log(jnp.sum(jnp.exp(x - m), axis=1, keepdims=True))

    lbl = lbl_ref[0, 0, :][:, None].astype(jnp.int32)
    p_mask = lbl <= G - 1
    tgt = jnp.where(p_mask, lbl, arg1)
    tval = jnp.sum(jnp.where(col == tgt, x, 0.0), axis=1, keepdims=True)
    nll = lse - tval

    u_mask = jnp.logical_not(p_mask)
    keys = jnp.where(u_mask, lax.bitcast_convert_type(diff, jnp.int32),
                     jnp.int32(-1))

    keys_s[pl.ds(i * BLK, BLK), :] = keys
    nll_s[pl.ds(i * BLK, BLK), :] = jnp.where(u_mask, nll, 0.0)

    accf[0] += jnp.sum(jnp.where(p_mask, nll, 0.0))
    acci[0] += jnp.sum(p_mask.astype(jnp.int32))
    acci[1] += jnp.sum(u_mask.astype(jnp.int32))

    @pl.when(i == NB - 1)
    def _finish():
        k = acci[1] // 10
        keys_all = keys_s[:, :]
        nll_all = nll_s[:, :]

        def bs(_, c):
            lo, hi = c
            mid = lo + (hi - lo) // 2
            cnt = jnp.sum((keys_all >= mid).astype(jnp.int32))
            ok = cnt >= k
            return jnp.where(ok, mid, lo), jnp.where(ok, hi, mid)

        tau, _ = lax.fori_loop(0, 31, bs, (jnp.int32(0), jnp.int32(HI_BITS)))

        gt = keys_all > tau
        cgt = jnp.sum(gt.astype(jnp.int32))
        need = k - cgt
        sum_gt = jnp.sum(jnp.where(gt, nll_all, 0.0))

        tie = keys_all == tau
        idx = lax.broadcasted_iota(jnp.int32, (N, 1), 0)

        def bs2(_, c):
            lo, hi = c
            mid = lo + (hi - lo + 1) // 2
            cnt = jnp.sum((tie & (idx < mid)).astype(jnp.int32))
            ok = cnt <= need
            return jnp.where(ok, mid, lo), jnp.where(ok, hi, mid - 1)

        mcut, _ = lax.fori_loop(0, 15, bs2, (jnp.int32(0), jnp.int32(N)))
        sum_tie = jnp.sum(jnp.where(tie & (idx < mcut), nll_all, 0.0))

        total = accf[0] + sum_gt + sum_tie
        cnt_all = acci[0] + k
        loss_ref[:, :] = jnp.broadcast_to(total / cnt_all.astype(jnp.float32),
                                          (1, 1))


@jax.jit
def _run(outputs, labels3, gpad):
    return pl.pallas_call(
        _body,
        grid=(NB,),
        in_specs=[
            pl.BlockSpec((BLK, D), lambda i: (i, 0)),
            pl.BlockSpec((1, 1, BLK), lambda i: (i, 0, 0)),
            pl.BlockSpec((D, D), lambda i: (0, 0)),
        ],
        out_specs=pl.BlockSpec((1, 1), lambda i: (0, 0)),
        out_shape=jax.ShapeDtypeStruct((1, 1), jnp.float32),
        scratch_shapes=[
            pltpu.VMEM((N, 1), jnp.int32),
            pltpu.VMEM((N, 1), jnp.float32),
            pltpu.SMEM((1,), jnp.float32),
            pltpu.SMEM((2,), jnp.int32),
        ],
        compiler_params=pltpu.CompilerParams(
            dimension_semantics=("arbitrary",),
        ),
    )(outputs, labels3, gpad)


def kernel(outputs, labels, global_logit):
    outputs = outputs.astype(jnp.float32)
    labels3 = labels.astype(jnp.int32).reshape(NB, 1, BLK)
    gpad = jnp.zeros((D, D), jnp.float32).at[:G].set(global_logit)
    return _run(outputs, labels3, gpad)[0, 0]


# speedup vs baseline: 8.9474x; 4.2940x over previous
import jax
import jax.numpy as jnp
from jax import lax
from jax.experimental import pallas as pl
from jax.experimental.pallas import tpu as pltpu

N = 16384
D = 128
G = 100
BLK = 512
NB = N // BLK
EPS = 1e-8
NEG = -1e30
HI_BITS = 0x40800000


def _body(out_ref, lbl_ref, g_ref, loss_ref, keys_s, nll_s, accf, acci):
    i = pl.program_id(0)

    @pl.when(i == 0)
    def _init():
        accf[0] = 0.0
        acci[0] = 0
        acci[1] = 0

    x = out_ref[:, :]
    g = g_ref[:, :]

    row = lax.broadcasted_iota(jnp.int32, (D, BLK), 0)

    gss = jnp.sum(g * g, axis=1, keepdims=True)
    gn = g / jnp.maximum(jnp.sqrt(gss), EPS)

    eye = (lax.broadcasted_iota(jnp.int32, (D, D), 0)
           == lax.broadcasted_iota(jnp.int32, (D, D), 1)).astype(jnp.float32)
    dn = (((1,), (1,)), ((), ()))
    xT = lax.dot_general(eye, x, dn, preferred_element_type=jnp.float32)
    simsT = lax.dot_general(gn, x, dn, preferred_element_type=jnp.float32)

    xss = jnp.sum(xT * xT, axis=0, keepdims=True)
    rinv = 1.0 / jnp.maximum(jnp.sqrt(xss), EPS)
    simsm = jnp.where(row < G, simsT * rinv, NEG)

    top1 = jnp.max(simsm, axis=0, keepdims=True)
    arg1 = jnp.min(jnp.where(simsm == top1, row, D), axis=0, keepdims=True)
    top2 = jnp.max(jnp.where(row == arg1, NEG, simsm), axis=0, keepdims=True)
    diff = top1 - top2

    m = jnp.max(xT, axis=0, keepdims=True)
    lse = m + jnp.log(jnp.sum(jnp.exp(xT - m), axis=0, keepdims=True))

    lbl = lbl_ref[0].astype(jnp.int32)
    p_mask = lbl <= G - 1
    tgt = jnp.where(p_mask, lbl, arg1)
    tval = jnp.sum(jnp.where(row == tgt, xT, 0.0), axis=0, keepdims=True)
    nll = lse - tval

    u_mask = jnp.logical_not(p_mask)
    keys = jnp.where(u_mask, lax.bitcast_convert_type(diff, jnp.int32),
                     jnp.int32(-1))

    keys_s[pl.ds(i, 1), :] = keys
    nll_s[pl.ds(i, 1), :] = jnp.where(u_mask, nll, 0.0)

    accf[0] += jnp.sum(jnp.where(p_mask, nll, 0.0))
    acci[0] += jnp.sum(p_mask.astype(jnp.int32))
    acci[1] += jnp.sum(u_mask.astype(jnp.int32))

    @pl.when(i == NB - 1)
    def _finish():
        k = acci[1] // 10
        keys_all = keys_s[:, :]
        nll_all = nll_s[:, :]

        def bs(_, c):
            lo, hi = c
            mid = lo + (hi - lo) // 2
            cnt = jnp.sum((keys_all >= mid).astype(jnp.int32))
            ok = cnt >= k
            return jnp.where(ok, mid, lo), jnp.where(ok, hi, mid)

        tau, _ = lax.fori_loop(0, 31, bs, (jnp.int32(0), jnp.int32(HI_BITS)))

        gt = keys_all > tau
        cgt = jnp.sum(gt.astype(jnp.int32))
        need = k - cgt
        sum_gt = jnp.sum(jnp.where(gt, nll_all, 0.0))

        tie = keys_all == tau
        idx = (lax.broadcasted_iota(jnp.int32, (NB, BLK), 0) * BLK
               + lax.broadcasted_iota(jnp.int32, (NB, BLK), 1))

        def bs2(_, c):
            lo, hi = c
            mid = lo + (hi - lo + 1) // 2
            cnt = jnp.sum((tie & (idx < mid)).astype(jnp.int32))
            ok = cnt <= need
            return jnp.where(ok, mid, lo), jnp.where(ok, hi, mid - 1)

        mcut, _ = lax.fori_loop(0, 15, bs2, (jnp.int32(0), jnp.int32(N)))
        sum_tie = jnp.sum(jnp.where(tie & (idx < mcut), nll_all, 0.0))

        total = accf[0] + sum_gt + sum_tie
        cnt_all = acci[0] + k
        loss_ref[:, :] = jnp.broadcast_to(total / cnt_all.astype(jnp.float32),
                                          (1, 1))


@jax.jit
def _run(outputs, labels3, gpad):
    return pl.pallas_call(
        _body,
        grid=(NB,),
        in_specs=[
            pl.BlockSpec((BLK, D), lambda i: (i, 0)),
            pl.BlockSpec((1, 1, BLK), lambda i: (i, 0, 0)),
            pl.BlockSpec((D, D), lambda i: (0, 0)),
        ],
        out_specs=pl.BlockSpec((1, 1), lambda i: (0, 0)),
        out_shape=jax.ShapeDtypeStruct((1, 1), jnp.float32),
        scratch_shapes=[
            pltpu.VMEM((NB, BLK), jnp.int32),
            pltpu.VMEM((NB, BLK), jnp.float32),
            pltpu.SMEM((1,), jnp.float32),
            pltpu.SMEM((2,), jnp.int32),
        ],
        compiler_params=pltpu.CompilerParams(
            dimension_semantics=("arbitrary",),
        ),
    )(outputs, labels3, gpad)


def kernel(outputs, labels, global_logit):
    outputs = outputs.astype(jnp.float32)
    labels3 = labels.astype(jnp.int32).reshape(NB, 1, BLK)
    gpad = jnp.zeros((D, D), jnp.float32).at[:G].set(global_logit)
    return _run(outputs, labels3, gpad)[0, 0]


# speedup vs baseline: 15.3627x; 1.7170x over previous
import jax
import jax.numpy as jnp
from jax import lax
from jax.experimental import pallas as pl
from jax.experimental.pallas import tpu as pltpu

N = 16384
D = 128
G = 100
BLK = 2048
NB = N // BLK
EPS = 1e-8
NEG = -1e30
HI_BITS = 0x40800000


def _body(out_ref, lbl_ref, g_ref, loss_ref, keys_s, nll_s, accf, acci):
    i = pl.program_id(0)

    @pl.when(i == 0)
    def _init():
        accf[0] = 0.0
        acci[0] = 0
        acci[1] = 0

    x = out_ref[:, :]
    g = g_ref[:, :]

    row = lax.broadcasted_iota(jnp.int32, (D, BLK), 0)

    gss = jnp.sum(g * g, axis=1, keepdims=True)
    gn = g / jnp.maximum(jnp.sqrt(gss), EPS)

    eye = (lax.broadcasted_iota(jnp.int32, (D, D), 0)
           == lax.broadcasted_iota(jnp.int32, (D, D), 1)).astype(jnp.float32)
    dn = (((1,), (1,)), ((), ()))
    xT = lax.dot_general(eye, x, dn, preferred_element_type=jnp.float32)
    simsT = lax.dot_general(gn, x, dn, preferred_element_type=jnp.float32)

    xss = jnp.sum(xT * xT, axis=0, keepdims=True)
    rinv = 1.0 / jnp.maximum(jnp.sqrt(xss), EPS)
    simsm = jnp.where(row < G, simsT * rinv, NEG)

    top1 = jnp.max(simsm, axis=0, keepdims=True)
    arg1 = jnp.min(jnp.where(simsm == top1, row, D), axis=0, keepdims=True)
    top2 = jnp.max(jnp.where(row == arg1, NEG, simsm), axis=0, keepdims=True)
    diff = top1 - top2

    m = jnp.max(xT, axis=0, keepdims=True)
    lse = m + jnp.log(jnp.sum(jnp.exp(xT - m), axis=0, keepdims=True))

    lbl = lbl_ref[0].astype(jnp.int32)
    p_mask = lbl <= G - 1
    tgt = jnp.where(p_mask, lbl, arg1)
    tval = jnp.sum(jnp.where(row == tgt, xT, 0.0), axis=0, keepdims=True)
    nll = lse - tval

    u_mask = jnp.logical_not(p_mask)
    keys = jnp.where(u_mask, lax.bitcast_convert_type(diff, jnp.int32),
                     jnp.int32(-1))

    keys_s[pl.ds(i, 1), :] = keys
    nll_s[pl.ds(i, 1), :] = jnp.where(u_mask, nll, 0.0)

    accf[0] += jnp.sum(jnp.where(p_mask, nll, 0.0))
    acci[0] += jnp.sum(p_mask.astype(jnp.int32))
    acci[1] += jnp.sum(u_mask.astype(jnp.int32))

    @pl.when(i == NB - 1)
    def _finish():
        k = acci[1] // 10
        keys_all = keys_s[:, :]
        nll_all = nll_s[:, :]

        def bs(_, c):
            lo, hi = c
            mid = lo + (hi - lo) // 2
            cnt = jnp.sum((keys_all >= mid).astype(jnp.int32))
            ok = cnt >= k
            return jnp.where(ok, mid, lo), jnp.where(ok, hi, mid)

        tau, _ = lax.fori_loop(0, 31, bs, (jnp.int32(0), jnp.int32(HI_BITS)))

        gt = keys_all > tau
        cgt = jnp.sum(gt.astype(jnp.int32))
        need = k - cgt
        sum_gt = jnp.sum(jnp.where(gt, nll_all, 0.0))

        tie = keys_all == tau
        idx = (lax.broadcasted_iota(jnp.int32, (NB, BLK), 0) * BLK
               + lax.broadcasted_iota(jnp.int32, (NB, BLK), 1))

        def bs2(_, c):
            lo, hi = c
            mid = lo + (hi - lo + 1) // 2
            cnt = jnp.sum((tie & (idx < mid)).astype(jnp.int32))
            ok = cnt <= need
            return jnp.where(ok, mid, lo), jnp.where(ok, hi, mid - 1)

        mcut, _ = lax.fori_loop(0, 15, bs2, (jnp.int32(0), jnp.int32(N)))
        sum_tie = jnp.sum(jnp.where(tie & (idx < mcut), nll_all, 0.0))

        total = accf[0] + sum_gt + sum_tie
        cnt_all = acci[0] + k
        loss_ref[:, :] = jnp.broadcast_to(total / cnt_all.astype(jnp.float32),
                                          (1, 1))


@jax.jit
def _run(outputs, labels3, gpad):
    return pl.pallas_call(
        _body,
        grid=(NB,),
        in_specs=[
            pl.BlockSpec((BLK, D), lambda i: (i, 0)),
            pl.BlockSpec((1, 1, BLK), lambda i: (i, 0, 0)),
            pl.BlockSpec((D, D), lambda i: (0, 0)),
        ],
        out_specs=pl.BlockSpec((1, 1), lambda i: (0, 0)),
        out_shape=jax.ShapeDtypeStruct((1, 1), jnp.float32),
        scratch_shapes=[
            pltpu.VMEM((NB, BLK), jnp.int32),
            pltpu.VMEM((NB, BLK), jnp.float32),
            pltpu.SMEM((1,), jnp.float32),
            pltpu.SMEM((2,), jnp.int32),
        ],
        compiler_params=pltpu.CompilerParams(
            dimension_semantics=("arbitrary",),
        ),
    )(outputs, labels3, gpad)


def kernel(outputs, labels, global_logit):
    outputs = outputs.astype(jnp.float32)
    labels3 = labels.astype(jnp.int32).reshape(NB, 1, BLK)
    gpad = jnp.zeros((D, D), jnp.float32).at[:G].set(global_logit)
    return _run(outputs, labels3, gpad)[0, 0]


# speedup vs baseline: 15.3814x; 1.0012x over previous
import jax
import jax.numpy as jnp
from jax import lax
from jax.experimental import pallas as pl
from jax.experimental.pallas import tpu as pltpu

N = 16384
D = 128
G = 100
BLK = 2048
NB = N // BLK
EPS = 1e-8
NEG = -1e30
HI_BITS = 0x40800000


def _body(out_ref, lbl_ref, g_ref, loss_ref, keys_s, nll_s, accf, acci):
    i = pl.program_id(0)

    @pl.when(i == 0)
    def _init():
        accf[0] = 0.0
        acci[0] = 0
        acci[1] = 0

    x = out_ref[:, :]
    g = g_ref[:, :]

    row = lax.broadcasted_iota(jnp.int32, (D, BLK), 0)

    gss = jnp.sum(g * g, axis=1, keepdims=True)
    gn = g / jnp.maximum(jnp.sqrt(gss), EPS)

    eye = (lax.broadcasted_iota(jnp.int32, (D, D), 0)
           == lax.broadcasted_iota(jnp.int32, (D, D), 1)).astype(jnp.float32)
    ones_r = jnp.ones((1, D), jnp.float32)
    dn = (((1,), (1,)), ((), ()))
    dn0 = (((1,), (0,)), ((), ()))
    xT = lax.dot_general(eye, x, dn, preferred_element_type=jnp.float32)
    simsT = lax.dot_general(gn, x, dn, preferred_element_type=jnp.float32)

    xss = lax.dot_general(ones_r, xT * xT, dn0,
                          preferred_element_type=jnp.float32)
    rinv = 1.0 / jnp.maximum(jnp.sqrt(xss), EPS)
    simsm = jnp.where(row < G, simsT * rinv, NEG)

    top1 = jnp.max(simsm, axis=0, keepdims=True)
    arg1 = jnp.min(jnp.where(simsm == top1, row, D), axis=0, keepdims=True)
    top2 = jnp.max(jnp.where(row == arg1, NEG, simsm), axis=0, keepdims=True)
    diff = top1 - top2

    m = jnp.max(xT, axis=0, keepdims=True)
    esum = lax.dot_general(ones_r, jnp.exp(xT - m), dn0,
                           preferred_element_type=jnp.float32)
    lse = m + jnp.log(esum)

    lbl = lbl_ref[0].astype(jnp.int32)
    p_mask = lbl <= G - 1
    tgt = jnp.where(p_mask, lbl, arg1)
    tval = lax.dot_general(ones_r, jnp.where(row == tgt, xT, 0.0), dn0,
                           preferred_element_type=jnp.float32)
    nll = lse - tval

    u_mask = jnp.logical_not(p_mask)
    keys = jnp.where(u_mask, lax.bitcast_convert_type(diff, jnp.int32),
                     jnp.int32(-1))

    keys_s[pl.ds(i, 1), :] = keys
    nll_s[pl.ds(i, 1), :] = jnp.where(u_mask, nll, 0.0)

    accf[0] += jnp.sum(jnp.where(p_mask, nll, 0.0))
    acci[0] += jnp.sum(p_mask.astype(jnp.int32))
    acci[1] += jnp.sum(u_mask.astype(jnp.int32))

    @pl.when(i == NB - 1)
    def _finish():
        k = acci[1] // 10
        keys_all = keys_s[:, :]
        nll_all = nll_s[:, :]

        def bs(_, c):
            lo, hi = c
            mid = lo + (hi - lo) // 2
            cnt = jnp.sum((keys_all >= mid).astype(jnp.int32), keepdims=True)
            ok = cnt >= k
            return jnp.where(ok, mid, lo), jnp.where(ok, hi, mid)

        tau, _ = lax.fori_loop(
            0, 31, bs,
            (jnp.zeros((1, 1), jnp.int32), jnp.full((1, 1), HI_BITS, jnp.int32)))

        gt = keys_all > tau
        cgt = jnp.sum(gt.astype(jnp.int32), keepdims=True)
        need = k - cgt
        sum_gt = jnp.sum(jnp.where(gt, nll_all, 0.0), keepdims=True)

        tie = keys_all == tau
        idx = (lax.broadcasted_iota(jnp.int32, (NB, BLK), 0) * BLK
               + lax.broadcasted_iota(jnp.int32, (NB, BLK), 1))

        def bs2(_, c):
            lo, hi = c
            mid = lo + (hi - lo + 1) // 2
            cnt = jnp.sum((tie & (idx < mid)).astype(jnp.int32), keepdims=True)
            ok = cnt <= need
            return jnp.where(ok, mid, lo), jnp.where(ok, hi, mid - 1)

        mcut, _ = lax.fori_loop(
            0, 15, bs2,
            (jnp.zeros((1, 1), jnp.int32), jnp.full((1, 1), N, jnp.int32)))
        sum_tie = jnp.sum(jnp.where(tie & (idx < mcut), nll_all, 0.0),
                          keepdims=True)

        total = accf[0] + sum_gt + sum_tie
        cnt_all = acci[0] + k
        loss_ref[:, :] = total / cnt_all.astype(jnp.float32)


@jax.jit
def _run(outputs, labels3, gpad):
    return pl.pallas_call(
        _body,
        grid=(NB,),
        in_specs=[
            pl.BlockSpec((BLK, D), lambda i: (i, 0)),
            pl.BlockSpec((1, 1, BLK), lambda i: (i, 0, 0)),
            pl.BlockSpec((D, D), lambda i: (0, 0)),
        ],
        out_specs=pl.BlockSpec((1, 1), lambda i: (0, 0)),
        out_shape=jax.ShapeDtypeStruct((1, 1), jnp.float32),
        scratch_shapes=[
            pltpu.VMEM((NB, BLK), jnp.int32),
            pltpu.VMEM((NB, BLK), jnp.float32),
            pltpu.SMEM((1,), jnp.float32),
            pltpu.SMEM((2,), jnp.int32),
        ],
        compiler_params=pltpu.CompilerParams(
            dimension_semantics=("arbitrary",),
        ),
    )(outputs, labels3, gpad)


def kernel(outputs, labels, global_logit):
    outputs = outputs.astype(jnp.float32)
    labels3 = labels.astype(jnp.int32).reshape(NB, 1, BLK)
    gpad = jnp.zeros((D, D), jnp.float32).at[:G].set(global_logit)
    return _run(outputs, labels3, gpad)[0, 0]


# speedup vs baseline: 15.4905x; 1.0071x over previous
import jax
import jax.numpy as jnp
from jax import lax
from jax.experimental import pallas as pl
from jax.experimental.pallas import tpu as pltpu

N = 16384
D = 128
G = 100
BLK = 4096
NB = N // BLK
EPS = 1e-8
NEG = -1e30
HI_BITS = 0x40800000


def _body(out_ref, lbl_ref, g_ref, loss_ref, keys_s, nll_s, accf, acci):
    i = pl.program_id(0)

    @pl.when(i == 0)
    def _init():
        accf[0] = 0.0
        acci[0] = 0
        acci[1] = 0

    x = out_ref[:, :]
    g = g_ref[:, :]

    row = lax.broadcasted_iota(jnp.int32, (D, BLK), 0)

    gss = jnp.sum(g * g, axis=1, keepdims=True)
    gn = g / jnp.maximum(jnp.sqrt(gss), EPS)

    eye = (lax.broadcasted_iota(jnp.int32, (D, D), 0)
           == lax.broadcasted_iota(jnp.int32, (D, D), 1)).astype(jnp.float32)
    ones_r = jnp.ones((1, D), jnp.float32)
    dn = (((1,), (1,)), ((), ()))
    dn0 = (((1,), (0,)), ((), ()))
    xT = lax.dot_general(eye, x, dn, preferred_element_type=jnp.float32)
    simsT = lax.dot_general(gn, x, dn, preferred_element_type=jnp.float32)

    xss = lax.dot_general(ones_r, xT * xT, dn0,
                          preferred_element_type=jnp.float32)
    rinv = 1.0 / jnp.maximum(jnp.sqrt(xss), EPS)
    simsm = jnp.where(row < G, simsT * rinv, NEG)

    top1 = jnp.max(simsm, axis=0, keepdims=True)
    arg1 = jnp.min(jnp.where(simsm == top1, row, D), axis=0, keepdims=True)
    top2 = jnp.max(jnp.where(row == arg1, NEG, simsm), axis=0, keepdims=True)
    diff = top1 - top2

    m = jnp.max(xT, axis=0, keepdims=True)
    esum = lax.dot_general(ones_r, jnp.exp(xT - m), dn0,
                           preferred_element_type=jnp.float32)
    lse = m + jnp.log(esum)

    lbl = lbl_ref[0].astype(jnp.int32)
    p_mask = lbl <= G - 1
    tgt = jnp.where(p_mask, lbl, arg1)
    tval = lax.dot_general(ones_r, jnp.where(row == tgt, xT, 0.0), dn0,
                           preferred_element_type=jnp.float32)
    nll = lse - tval

    u_mask = jnp.logical_not(p_mask)
    keys = jnp.where(u_mask, lax.bitcast_convert_type(diff, jnp.int32),
                     jnp.int32(-1))

    keys_s[pl.ds(i, 1), :] = keys
    nll_s[pl.ds(i, 1), :] = jnp.where(u_mask, nll, 0.0)

    accf[0] += jnp.sum(jnp.where(p_mask, nll, 0.0))
    acci[0] += jnp.sum(p_mask.astype(jnp.int32))
    acci[1] += jnp.sum(u_mask.astype(jnp.int32))

    @pl.when(i == NB - 1)
    def _finish():
        k = acci[1] // 10
        keys_all = keys_s[:, :]
        nll_all = nll_s[:, :]

        def bs(_, c):
            lo, hi = c
            mid = lo + (hi - lo) // 2
            cnt = jnp.sum((keys_all >= mid).astype(jnp.int32), keepdims=True)
            ok = cnt >= k
            return jnp.where(ok, mid, lo), jnp.where(ok, hi, mid)

        tau, _ = lax.fori_loop(
            0, 31, bs,
            (jnp.zeros((1, 1), jnp.int32), jnp.full((1, 1), HI_BITS, jnp.int32)))

        gt = keys_all > tau
        cgt = jnp.sum(gt.astype(jnp.int32), keepdims=True)
        need = k - cgt
        sum_gt = jnp.sum(jnp.where(gt, nll_all, 0.0), keepdims=True)

        tie = keys_all == tau
        idx = (lax.broadcasted_iota(jnp.int32, (NB, BLK), 0) * BLK
               + lax.broadcasted_iota(jnp.int32, (NB, BLK), 1))

        def bs2(_, c):
            lo, hi = c
            mid = lo + (hi - lo + 1) // 2
            cnt = jnp.sum((tie & (idx < mid)).astype(jnp.int32), keepdims=True)
            ok = cnt <= need
            return jnp.where(ok, mid, lo), jnp.where(ok, hi, mid - 1)

        mcut, _ = lax.fori_loop(
            0, 15, bs2,
            (jnp.zeros((1, 1), jnp.int32), jnp.full((1, 1), N, jnp.int32)))
        sum_tie = jnp.sum(jnp.where(tie & (idx < mcut), nll_all, 0.0),
                          keepdims=True)

        total = accf[0] + sum_gt + sum_tie
        cnt_all = acci[0] + k
        loss_ref[:, :] = total / cnt_all.astype(jnp.float32)


@jax.jit
def _run(outputs, labels3, gpad):
    return pl.pallas_call(
        _body,
        grid=(NB,),
        in_specs=[
            pl.BlockSpec((BLK, D), lambda i: (i, 0)),
            pl.BlockSpec((1, 1, BLK), lambda i: (i, 0, 0)),
            pl.BlockSpec((D, D), lambda i: (0, 0)),
        ],
        out_specs=pl.BlockSpec((1, 1), lambda i: (0, 0)),
        out_shape=jax.ShapeDtypeStruct((1, 1), jnp.float32),
        scratch_shapes=[
            pltpu.VMEM((NB, BLK), jnp.int32),
            pltpu.VMEM((NB, BLK), jnp.float32),
            pltpu.SMEM((1,), jnp.float32),
            pltpu.SMEM((2,), jnp.int32),
        ],
        compiler_params=pltpu.CompilerParams(
            dimension_semantics=("arbitrary",),
        ),
    )(outputs, labels3, gpad)


def kernel(outputs, labels, global_logit):
    outputs = outputs.astype(jnp.float32)
    labels3 = labels.astype(jnp.int32).reshape(NB, 1, BLK)
    gpad = jnp.zeros((D, D), jnp.float32).at[:G].set(global_logit)
    return _run(outputs, labels3, gpad)[0, 0]
